# baseline (speedup 1.0000x reference)
import functools

import jax
import jax.numpy as jnp
from jax import lax
from jax.experimental import pallas as pl
from jax.experimental.pallas import tpu as pltpu
from jax.experimental.pallas import tpu_sc as plsc

_NC = 2
_NS = 16
_NW = _NC * _NS
_RPW = 8
_LANES = 16


def _embed_sum_body(active, embed, idx_hbm, emb_hbm, out_hbm,
                    idx_v, rows_v, acc_v, all_v, shared, sem):
    c = lax.axis_index("c")
    s = lax.axis_index("s")
    wid = s * _NC + c
    base = wid * _RPW
    pltpu.sync_copy(idx_hbm.at[pl.ds(base, _RPW)], idx_v)
    pltpu.async_copy(emb_hbm.at[idx_v], rows_v, sem).wait()
    scale = jnp.where(wid < active, jnp.float32(1), jnp.float32(0))
    for ch in range(embed // _LANES):
        v = rows_v[0, pl.ds(ch * _LANES, _LANES)]
        for r in range(1, _RPW):
            v = v + rows_v[r, pl.ds(ch * _LANES, _LANES)]
        acc_v[pl.ds(ch * _LANES, _LANES)] = v * scale
    pltpu.sync_copy(acc_v, shared.at[s])
    plsc.subcore_barrier()

    @pl.when(s == 0)
    def _():
        pltpu.sync_copy(shared, all_v)
        for ch in range(embed // _LANES):
            v = all_v[0, pl.ds(ch * _LANES, _LANES)]
            for w in range(1, _NS):
                v = v + all_v[w, pl.ds(ch * _LANES, _LANES)]
            acc_v[pl.ds(ch * _LANES, _LANES)] = v
        pltpu.sync_copy(acc_v, out_hbm.at[c])


def _embed_sum_sc(idx_pad, emb):
    embed = emb.shape[1]
    active = idx_pad.shape[0] // _RPW
    mesh = plsc.VectorSubcoreMesh(
        core_axis_name="c", subcore_axis_name="s",
        num_cores=_NC, num_subcores=_NS)
    padded = jnp.concatenate(
        [idx_pad, jnp.zeros(((_NW * _RPW) - idx_pad.shape[0],), jnp.int32)])
    kern = pl.kernel(
        functools.partial(_embed_sum_body, active, embed),
        out_type=jax.ShapeDtypeStruct((_NC, embed), jnp.float32),
        mesh=mesh,
        scratch_types=[
            pltpu.VMEM((_RPW,), jnp.int32),
            pltpu.VMEM((_RPW, embed), jnp.float32),
            pltpu.VMEM((embed,), jnp.float32),
            pltpu.VMEM((_NS, embed), jnp.float32),
            pltpu.VMEM_SHARED((_NS, embed), jnp.float32),
            pltpu.SemaphoreType.DMA,
        ],
    )
    return kern(padded, emb)


_TILE = 2048


def _matvec_body(e_ref, w_ref, b_ref, o_ref):
    e = e_ref[0:1, :] + e_ref[1:2, :]
    o_ref[...] = jax.lax.dot_general(
        e, w_ref[...],
        dimension_numbers=(((1,), (1,)), ((), ())),
        preferred_element_type=jnp.float32) + b_ref[...]


def _matvec_tc(partials, W, b2):
    vocab, embed = W.shape
    grid = (vocab + _TILE - 1) // _TILE
    return pl.pallas_call(
        _matvec_body,
        grid=(grid,),
        in_specs=[
            pl.BlockSpec((_NC, embed), lambda i: (0, 0)),
            pl.BlockSpec((_TILE, embed), lambda i: (i, 0)),
            pl.BlockSpec((1, _TILE), lambda i: (0, i)),
        ],
        out_specs=pl.BlockSpec((1, _TILE), lambda i: (0, i)),
        out_shape=jax.ShapeDtypeStruct((1, vocab), jnp.float32),
    )(partials, W, b2)


def kernel(inputs, emb, W, b):
    idx = inputs.astype(jnp.int32)
    assert idx.shape[0] % _RPW == 0
    partials = _embed_sum_sc(idx, emb)
    return _matvec_tc(partials, W, b.reshape(1, -1))


# speedup vs baseline: 1.0832x; 1.0832x over previous
import functools

import jax
import jax.numpy as jnp
from jax import lax
from jax.experimental import pallas as pl
from jax.experimental.pallas import tpu as pltpu
from jax.experimental.pallas import tpu_sc as plsc

_NC = 2
_NS = 16
_NW = _NC * _NS
_RPW = 8
_LANES = 16


def _embed_sum_body(active, embed, idx_hbm, emb_hbm, out_hbm,
                    idx_v, rows_v, acc_v, all_v, shared, sem):
    c = lax.axis_index("c")
    s = lax.axis_index("s")
    wid = s * _NC + c
    base = wid * _RPW
    pltpu.sync_copy(idx_hbm.at[pl.ds(base, _RPW)], idx_v)
    pltpu.async_copy(emb_hbm.at[idx_v], rows_v, sem).wait()
    scale = jnp.where(wid < active, jnp.float32(1), jnp.float32(0))
    for ch in range(embed // _LANES):
        v = rows_v[0, pl.ds(ch * _LANES, _LANES)]
        for r in range(1, _RPW):
            v = v + rows_v[r, pl.ds(ch * _LANES, _LANES)]
        acc_v[pl.ds(ch * _LANES, _LANES)] = v * scale
    pltpu.sync_copy(acc_v, shared.at[s])
    plsc.subcore_barrier()

    @pl.when(s == 0)
    def _():
        pltpu.sync_copy(shared, all_v)
        for ch in range(embed // _LANES):
            v = all_v[0, pl.ds(ch * _LANES, _LANES)]
            for w in range(1, _NS):
                v = v + all_v[w, pl.ds(ch * _LANES, _LANES)]
            acc_v[pl.ds(ch * _LANES, _LANES)] = v
        pltpu.sync_copy(acc_v, out_hbm.at[c])


def _embed_sum_sc(idx_pad, emb):
    embed = emb.shape[1]
    active = idx_pad.shape[0] // _RPW
    mesh = plsc.VectorSubcoreMesh(
        core_axis_name="c", subcore_axis_name="s",
        num_cores=_NC, num_subcores=_NS)
    padded = jnp.concatenate(
        [idx_pad, jnp.zeros(((_NW * _RPW) - idx_pad.shape[0],), jnp.int32)])
    kern = pl.kernel(
        functools.partial(_embed_sum_body, active, embed),
        out_type=jax.ShapeDtypeStruct((_NC, embed), jnp.float32),
        mesh=mesh,
        scratch_types=[
            pltpu.VMEM((_RPW,), jnp.int32),
            pltpu.VMEM((_RPW, embed), jnp.float32),
            pltpu.VMEM((embed,), jnp.float32),
            pltpu.VMEM((_NS, embed), jnp.float32),
            pltpu.VMEM_SHARED((_NS, embed), jnp.float32),
            pltpu.SemaphoreType.DMA,
        ],
    )
    return kern(padded, emb)


_TILE = 2048


def _matvec_body(e_ref, w_ref, b_ref, o_ref):
    e = e_ref[0:1, :] + e_ref[1:2, :]
    o_ref[...] = jax.lax.dot_general(
        e, w_ref[...],
        dimension_numbers=(((1,), (1,)), ((), ())),
        preferred_element_type=jnp.float32) + b_ref[...]


def _matvec_tc(partials, W, b2):
    vocab, embed = W.shape
    grid = (vocab + _TILE - 1) // _TILE
    return pl.pallas_call(
        _matvec_body,
        grid=(grid,),
        in_specs=[
            pl.BlockSpec((_NC, embed), lambda i: (0, 0)),
            pl.BlockSpec((_TILE, embed), lambda i: (i, 0)),
            pl.BlockSpec((1, _TILE), lambda i: (0, i)),
        ],
        out_specs=pl.BlockSpec((1, _TILE), lambda i: (0, i)),
        out_shape=jax.ShapeDtypeStruct((1, vocab), jnp.float32),
    )(partials, W, b2)


def kernel(inputs, emb, W, b):
    idx = inputs.astype(jnp.int32)
    assert idx.shape[0] % _RPW == 0
    e = jnp.take(emb, idx, axis=0).sum(axis=0)
    partials = jnp.stack([e, jnp.zeros_like(e)])
    return _matvec_tc(partials, W, b.reshape(1, -1))


# speedup vs baseline: 1.5026x; 1.3871x over previous
import functools

import jax
import jax.numpy as jnp
from jax import lax
from jax.experimental import pallas as pl
from jax.experimental.pallas import tpu as pltpu
from jax.experimental.pallas import tpu_sc as plsc

_NC = 2
_NS = 16
_NW = _NC * _NS
_RPW = 8
_LANES = 16


def _embed_sum_body(active, embed, idx_hbm, emb_hbm, out_hbm,
                    idx_v, rows_v, acc_v, all_v, shared, sem):
    c = lax.axis_index("c")
    s = lax.axis_index("s")
    wid = s * _NC + c
    base = wid * _RPW
    pltpu.sync_copy(idx_hbm.at[pl.ds(base, _RPW)], idx_v)
    pltpu.async_copy(emb_hbm.at[idx_v], rows_v, sem).wait()
    scale = jnp.where(wid < active, jnp.float32(1), jnp.float32(0))
    for ch in range(embed // _LANES):
        v = rows_v[0, pl.ds(ch * _LANES, _LANES)]
        for r in range(1, _RPW):
            v = v + rows_v[r, pl.ds(ch * _LANES, _LANES)]
        acc_v[pl.ds(ch * _LANES, _LANES)] = v * scale
    pltpu.sync_copy(acc_v, shared.at[s])
    plsc.subcore_barrier()

    @pl.when(s == 0)
    def _():
        pltpu.sync_copy(shared, all_v)
        for ch in range(embed // _LANES):
            v = all_v[0, pl.ds(ch * _LANES, _LANES)]
            for w in range(1, _NS):
                v = v + all_v[w, pl.ds(ch * _LANES, _LANES)]
            acc_v[pl.ds(ch * _LANES, _LANES)] = v
        pltpu.sync_copy(acc_v, out_hbm.at[c])


def _embed_sum_sc(idx_pad, emb):
    embed = emb.shape[1]
    active = idx_pad.shape[0] // _RPW
    mesh = plsc.VectorSubcoreMesh(
        core_axis_name="c", subcore_axis_name="s",
        num_cores=_NC, num_subcores=_NS)
    padded = jnp.concatenate(
        [idx_pad, jnp.zeros(((_NW * _RPW) - idx_pad.shape[0],), jnp.int32)])
    kern = pl.kernel(
        functools.partial(_embed_sum_body, active, embed),
        out_type=jax.ShapeDtypeStruct((_NC, embed), jnp.float32),
        mesh=mesh,
        scratch_types=[
            pltpu.VMEM((_RPW,), jnp.int32),
            pltpu.VMEM((_RPW, embed), jnp.float32),
            pltpu.VMEM((embed,), jnp.float32),
            pltpu.VMEM((_NS, embed), jnp.float32),
            pltpu.VMEM_SHARED((_NS, embed), jnp.float32),
            pltpu.SemaphoreType.DMA,
        ],
    )
    return kern(padded, emb)


_TILE = 8192


def _matvec_body(e_ref, w_ref, b_ref, o_ref):
    e = e_ref[0:1, :] + e_ref[1:2, :]
    o_ref[...] = jax.lax.dot_general(
        e, w_ref[...],
        dimension_numbers=(((1,), (1,)), ((), ())),
        preferred_element_type=jnp.float32) + b_ref[...]


def _matvec_tc(partials, W, b2):
    vocab, embed = W.shape
    grid = (vocab + _TILE - 1) // _TILE
    return pl.pallas_call(
        _matvec_body,
        grid=(grid,),
        in_specs=[
            pl.BlockSpec((_NC, embed), lambda i: (0, 0)),
            pl.BlockSpec((_TILE, embed), lambda i: (i, 0)),
            pl.BlockSpec((1, _TILE), lambda i: (0, i)),
        ],
        out_specs=pl.BlockSpec((1, _TILE), lambda i: (0, i)),
        out_shape=jax.ShapeDtypeStruct((1, vocab), jnp.float32),
    )(partials, W, b2)


def kernel(inputs, emb, W, b):
    idx = inputs.astype(jnp.int32)
    assert idx.shape[0] % _RPW == 0
    e = jnp.take(emb, idx, axis=0).sum(axis=0)
    partials = jnp.stack([e, jnp.zeros_like(e)])
    return _matvec_tc(partials, W, b.reshape(1, -1))


# speedup vs baseline: 1.5935x; 1.0605x over previous
import functools

import jax
import jax.numpy as jnp
from jax import lax
from jax.experimental import pallas as pl
from jax.experimental.pallas import tpu as pltpu
from jax.experimental.pallas import tpu_sc as plsc

_NC = 2
_NS = 16
_NW = _NC * _NS
_RPW = 8
_LANES = 16


def _embed_sum_body(active, embed, idx_hbm, emb_hbm, out_hbm,
                    idx_v, rows_v, acc_v, all_v, shared, sem):
    c = lax.axis_index("c")
    s = lax.axis_index("s")
    wid = s * _NC + c
    base = wid * _RPW
    pltpu.sync_copy(idx_hbm.at[pl.ds(base, _RPW)], idx_v)
    pltpu.async_copy(emb_hbm.at[idx_v], rows_v, sem).wait()
    scale = jnp.where(wid < active, jnp.float32(1), jnp.float32(0))
    for ch in range(embed // _LANES):
        v = rows_v[0, pl.ds(ch * _LANES, _LANES)]
        for r in range(1, _RPW):
            v = v + rows_v[r, pl.ds(ch * _LANES, _LANES)]
        acc_v[pl.ds(ch * _LANES, _LANES)] = v * scale
    pltpu.sync_copy(acc_v, shared.at[s])
    plsc.subcore_barrier()

    @pl.when(s == 0)
    def _():
        pltpu.sync_copy(shared, all_v)
        for ch in range(embed // _LANES):
            v = all_v[0, pl.ds(ch * _LANES, _LANES)]
            for w in range(1, _NS):
                v = v + all_v[w, pl.ds(ch * _LANES, _LANES)]
            acc_v[pl.ds(ch * _LANES, _LANES)] = v
        pltpu.sync_copy(acc_v, out_hbm.at[c])


def _embed_sum_sc(idx_pad, emb):
    embed = emb.shape[1]
    active = idx_pad.shape[0] // _RPW
    mesh = plsc.VectorSubcoreMesh(
        core_axis_name="c", subcore_axis_name="s",
        num_cores=_NC, num_subcores=_NS)
    padded = jnp.concatenate(
        [idx_pad, jnp.zeros(((_NW * _RPW) - idx_pad.shape[0],), jnp.int32)])
    kern = pl.kernel(
        functools.partial(_embed_sum_body, active, embed),
        out_type=jax.ShapeDtypeStruct((_NC, embed), jnp.float32),
        mesh=mesh,
        scratch_types=[
            pltpu.VMEM((_RPW,), jnp.int32),
            pltpu.VMEM((_RPW, embed), jnp.float32),
            pltpu.VMEM((embed,), jnp.float32),
            pltpu.VMEM((_NS, embed), jnp.float32),
            pltpu.VMEM_SHARED((_NS, embed), jnp.float32),
            pltpu.SemaphoreType.DMA,
        ],
    )
    return kern(padded, emb)


_TILE = 16384


def _matvec_body(e_ref, w_ref, b_ref, o_ref):
    e = e_ref[0:1, :] + e_ref[1:2, :]
    o_ref[...] = jax.lax.dot_general(
        e, w_ref[...],
        dimension_numbers=(((1,), (1,)), ((), ())),
        preferred_element_type=jnp.float32) + b_ref[...]


def _matvec_tc(partials, W, b2):
    vocab, embed = W.shape
    grid = (vocab + _TILE - 1) // _TILE
    return pl.pallas_call(
        _matvec_body,
        grid=(grid,),
        in_specs=[
            pl.BlockSpec((_NC, embed), lambda i: (0, 0)),
            pl.BlockSpec((_TILE, embed), lambda i: (i, 0)),
            pl.BlockSpec((1, _TILE), lambda i: (0, i)),
        ],
        out_specs=pl.BlockSpec((1, _TILE), lambda i: (0, i)),
        out_shape=jax.ShapeDtypeStruct((1, vocab), jnp.float32),
    )(partials, W, b2)


def kernel(inputs, emb, W, b):
    idx = inputs.astype(jnp.int32)
    assert idx.shape[0] % _RPW == 0
    e = jnp.take(emb, idx, axis=0).sum(axis=0)
    partials = jnp.stack([e, jnp.zeros_like(e)])
    return _matvec_tc(partials, W, b.reshape(1, -1))
